# Initial kernel scaffold; baseline (speedup 1.0000x reference)
import jax
import jax.numpy as jnp
from jax.experimental import pallas as pl


def kernel(coord, numbers, charge, afv, comb_v_a, comb_v_q, m0_w1, m0_b1, m0_w2, m0_b2, m1_w1, m1_b1, m1_w2, m1_b2, m2_w1, m2_b1, m2_w2, m2_b2):
    raise NotImplementedError("write your pallas kernel here")


# speedup vs baseline: 1.0452x; 1.0452x over previous
import jax
import jax.numpy as jnp
from jax.experimental import pallas as pl

NFEATURE = 32
NSHIFTS = 16
NCOMB_V = 8
RC = 5.0
B, N = 128, 64


def _fused(coord, coordt, numbers, charge, afv, cva, cvq,
           w01, b01, w02, b02, w11, b11, w12, b12, w21, b21, w22, b22,
           ch_out, aim_out):
    f32 = jnp.float32
    ncol = numbers[0].T
    iota_n = jax.lax.broadcasted_iota(jnp.int32, (N, 64), 1)
    oh = (ncol == iota_n).astype(f32)
    a = jnp.dot(oh, afv[...], preferred_element_type=f32)
    keep = 1.0 - oh[:, :1]

    rij = [coord[0][:, d:d + 1] - coordt[0][d:d + 1, :] for d in range(3)]
    dist = jnp.sqrt(rij[0] * rij[0] + rij[1] * rij[1]
                    + rij[2] * rij[2] + 1e-12)
    ii = jax.lax.broadcasted_iota(jnp.int32, (N, N), 0)
    jj = jax.lax.broadcasted_iota(jnp.int32, (N, N), 1)
    valid = ((ncol != 0) & (numbers[0] != 0) & (ii != jj) & (dist < RC))
    fc = 0.5 * jnp.cos(jnp.pi * jnp.clip(dist, 0.0, RC) / RC) + 0.5
    fc = jnp.where(valid, fc, 0.0)
    dsafe = jnp.where(valid, dist, 1.0)
    u = jnp.stack([jnp.where(valid, r / dsafe, 0.0) for r in rij], 0)

    s_io = jax.lax.broadcasted_iota(jnp.int32, (NSHIFTS, 1, 1), 0).astype(f32)
    shifts = 0.8 + s_io * ((RC - 0.8) / (NSHIFTS - 1))
    GS2 = jnp.exp(-4.0 * (dist[None] - shifts) ** 2) * fc[None]
    gs_isj = GS2.transpose(1, 0, 2)
    GS2f = GS2.reshape(NSHIFTS, N * N)
    gka = jnp.dot(cva[...].T, GS2f, preferred_element_type=f32
                  ).reshape(NCOMB_V, N, N).transpose(1, 0, 2)
    gkq = jnp.dot(cvq[...].T, GS2f, preferred_element_type=f32
                  ).reshape(NCOMB_V, N, N).transpose(1, 0, 2)
    ut = u.transpose(1, 0, 2)
    ugka = ut[:, :, None, :] * gka[:, None, :, :]
    ugkq = ut[:, :, None, :] * gkq[:, None, :, :]
    LHS = jnp.concatenate([gs_isj.reshape(N * NSHIFTS, N),
                           ugka.reshape(N * 24, N),
                           ugkq.reshape(N * 24, N)], 0)

    Q = charge[0]

    def conv_a_feats(prod):
        avs_t = prod[:N * NSHIFTS].reshape(N, NSHIFTS, -1).transpose(0, 2, 1)
        avf_s = avs_t[:, :NFEATURE, :].reshape(N, NFEATURE * NSHIFTS)
        va = prod[N * NSHIFTS:].reshape(N, 3, NCOMB_V, -1)[..., :NFEATURE]
        avf_v = (va * va).sum(1).transpose(0, 2, 1).reshape(N, NFEATURE * NCOMB_V)
        return avf_s, avf_v, avs_t

    def nqe(q, f):
        w = f * f
        w = w / (w.sum(0, keepdims=True) + 1e-6)
        return q + (Q - q.sum(0, keepdims=True)) * w

    prod = jnp.dot(LHS[:2560], a, preferred_element_type=f32)
    avf_s, avf_v, _ = conv_a_feats(prod)
    xin = jnp.concatenate([a, avf_s, avf_v], 1)
    h = jax.nn.gelu(jnp.dot(xin, w01[...], preferred_element_type=f32) + b01[...])
    out = (jnp.dot(h, w02[...], preferred_element_type=f32) + b02[...]) * keep
    charges = nqe(out[:, 0:1], out[:, 1:2])
    a = a + out[:, 2:]

    def pass_aq(a, charges, w1, b1, w2, b2):
        rhs = jnp.concatenate([a, charges], 1)
        prod = jnp.dot(LHS, rhs, preferred_element_type=f32)
        avf_s, avf_v, avs_t = conv_a_feats(prod[:2560])
        sq = avs_t[:, NFEATURE, :]
        vq = prod[2560:].reshape(N, 3, NCOMB_V, 33
                                 ).transpose(0, 1, 3, 2)[:, :, NFEATURE, :]
        avf_vq = (vq * vq).sum(1)
        xin = jnp.concatenate([a, avf_s, avf_v, charges, sq, avf_vq], 1)
        h = jax.nn.gelu(jnp.dot(xin, w1, preferred_element_type=f32) + b1)
        return jax.nn.gelu(jnp.dot(h, w2, preferred_element_type=f32) + b2) * keep

    out = pass_aq(a, charges, w11[...], b11[...], w12[...], b12[...])
    charges = nqe(charges + out[:, 0:1], out[:, 1:2])
    a = a + out[:, 2:]

    aim = pass_aq(a, charges, w21[...], b21[...], w22[...], b22[...])
    ch_out[0] = charges
    aim_out[0] = aim


def kernel(coord, numbers, charge, afv, comb_v_a, comb_v_q,
           m0_w1, m0_b1, m0_w2, m0_b2,
           m1_w1, m1_b1, m1_w2, m1_b2,
           m2_w1, m2_b1, m2_w2, m2_b2):
    f32 = jnp.float32
    coordt = coord.transpose(0, 2, 1)
    charge3 = charge.reshape(B, 1, 1)
    numbers3 = numbers.astype(jnp.int32).reshape(B, 1, N)

    full = lambda shp: pl.BlockSpec(shp, lambda i: (0,) * len(shp))
    in_specs = [
        pl.BlockSpec((1, N, 3), lambda i: (i, 0, 0)),
        pl.BlockSpec((1, 3, N), lambda i: (i, 0, 0)),
        pl.BlockSpec((1, 1, N), lambda i: (i, 0, 0)),
        pl.BlockSpec((1, 1, 1), lambda i: (i, 0, 0)),
        full((64, NFEATURE)),
        full((NSHIFTS, NCOMB_V)),
        full((NSHIFTS, NCOMB_V)),
        full(m0_w1.shape), full((1, 256)), full(m0_w2.shape), full((1, 34)),
        full(m1_w1.shape), full((1, 256)), full(m1_w2.shape), full((1, 34)),
        full(m2_w1.shape), full((1, 256)), full(m2_w2.shape), full((1, 256)),
    ]
    out_specs = [
        pl.BlockSpec((1, N, 1), lambda i: (i, 0, 0)),
        pl.BlockSpec((1, N, 256), lambda i: (i, 0, 0)),
    ]
    ch, aim = pl.pallas_call(
        _fused,
        grid=(B,),
        in_specs=in_specs,
        out_specs=out_specs,
        out_shape=[jax.ShapeDtypeStruct((B, N, 1), f32),
                   jax.ShapeDtypeStruct((B, N, 256), f32)],
    )(coord, coordt, numbers3, charge3, afv, comb_v_a, comb_v_q,
      m0_w1, m0_b1.reshape(1, -1), m0_w2, m0_b2.reshape(1, -1),
      m1_w1, m1_b1.reshape(1, -1), m1_w2, m1_b2.reshape(1, -1),
      m2_w1, m2_b1.reshape(1, -1), m2_w2, m2_b2.reshape(1, -1))
    return jnp.concatenate([ch, aim], -1)


# speedup vs baseline: 1.8763x; 1.7951x over previous
import jax
import jax.numpy as jnp
from jax.experimental import pallas as pl
from jax.experimental.pallas import tpu as pltpu

NFEATURE = 32
NSHIFTS = 16
NCOMB_V = 8
RC = 5.0
B, N = 128, 64
HID = 256


def _perm_w1_a(w1):
    a_part = w1[0:32]
    xs = w1[32:544].reshape(NFEATURE, NSHIFTS, HID).transpose(1, 0, 2)
    sv = w1[544:800].reshape(NFEATURE, NCOMB_V, HID).transpose(1, 0, 2)
    return jnp.concatenate([a_part, xs.reshape(512, HID),
                            sv.reshape(256, HID)], 0)


def _perm_w1_aq(w1):
    z1 = jnp.zeros((NCOMB_V, 1, HID), w1.dtype)
    a_part = w1[0:32]
    xs = jnp.concatenate(
        [w1[32:544].reshape(NFEATURE, NSHIFTS, HID).transpose(1, 0, 2),
         w1[801:817][:, None, :]], 1)
    sv = jnp.concatenate(
        [w1[544:800].reshape(NFEATURE, NCOMB_V, HID).transpose(1, 0, 2),
         z1], 1)
    svq = jnp.concatenate(
        [jnp.zeros((NCOMB_V, NFEATURE, HID), w1.dtype),
         w1[817:825][:, None, :]], 1)
    return jnp.concatenate([a_part, xs.reshape(528, HID), sv.reshape(264, HID),
                            w1[800:801], svq.reshape(264, HID)], 0)


def _fused(coord, coordt, numbers, charge, afv, cva, cvq,
           w01, b01, w02, b02, w11, b11, w12, b12, w21, b21, w22, b22,
           ch_out, aim_out):
    f32 = jnp.float32
    ncol = numbers[0].T
    iota_n = jax.lax.broadcasted_iota(jnp.int32, (N, 64), 1)
    oh = (ncol == iota_n).astype(f32)
    a = jnp.dot(oh, afv[...], preferred_element_type=f32)
    keep = 1.0 - oh[:, :1]

    rij = [coord[0][:, d:d + 1] - coordt[0][d:d + 1, :] for d in range(3)]
    dist = jnp.sqrt(rij[0] * rij[0] + rij[1] * rij[1]
                    + rij[2] * rij[2] + 1e-12)
    ii = jax.lax.broadcasted_iota(jnp.int32, (N, N), 0)
    jj = jax.lax.broadcasted_iota(jnp.int32, (N, N), 1)
    valid = ((ncol != 0) & (numbers[0] != 0) & (ii != jj) & (dist < RC))
    fc = 0.5 * jnp.cos(jnp.pi * jnp.clip(dist, 0.0, RC) / RC) + 0.5
    fc = jnp.where(valid, fc, 0.0)
    dsafe = jnp.where(valid, dist, 1.0)
    u = jnp.stack([jnp.where(valid, r / dsafe, 0.0) for r in rij], 0)

    s_io = jax.lax.broadcasted_iota(jnp.int32, (NSHIFTS, 1, 1), 0).astype(f32)
    shifts = 0.8 + s_io * ((RC - 0.8) / (NSHIFTS - 1))
    GS2 = jnp.exp(-4.0 * (dist[None] - shifts) ** 2) * fc[None]
    GS2f = GS2.reshape(NSHIFTS, N * N)
    gka = jnp.dot(cva[...].T, GS2f, preferred_element_type=f32
                  ).reshape(NCOMB_V, N, N)
    gkq = jnp.dot(cvq[...].T, GS2f, preferred_element_type=f32
                  ).reshape(NCOMB_V, N, N)
    ugka = u[:, None, :, :] * gka[None, :, :, :]
    ugkq = u[:, None, :, :] * gkq[None, :, :, :]
    LHS = jnp.concatenate([GS2.reshape(N * NSHIFTS, N),
                           ugka.reshape(N * 24, N),
                           ugkq.reshape(N * 24, N)], 0)

    Q = charge[0]

    def lane_blocks(mat, nblk):
        return jnp.concatenate([mat[t * N:(t + 1) * N] for t in range(nblk)], 1)

    def vpart(mat):
        sq = (mat * mat).reshape(3, NCOMB_V * N, -1).sum(0)
        return lane_blocks(sq, NCOMB_V)

    def nqe(q, f):
        w = f * f
        w = w / (w.sum(0, keepdims=True) + 1e-6)
        return q + (Q - q.sum(0, keepdims=True)) * w

    prod = jnp.dot(LHS[:2560], a, preferred_element_type=f32)
    xin = jnp.concatenate([a, lane_blocks(prod[:1024], NSHIFTS),
                           vpart(prod[1024:2560])], 1)
    h = jax.nn.gelu(jnp.dot(xin, w01[...], preferred_element_type=f32) + b01[...])
    out = (jnp.dot(h, w02[...], preferred_element_type=f32) + b02[...]) * keep
    charges = nqe(out[:, 0:1], out[:, 1:2])
    a = a + out[:, 2:]

    def pass_aq(a, charges, w1, b1, w2, b2):
        rhs = jnp.concatenate([a, charges], 1)
        prod = jnp.dot(LHS, rhs, preferred_element_type=f32)
        xin = jnp.concatenate([a, lane_blocks(prod[:1024], NSHIFTS),
                               vpart(prod[1024:2560]), charges,
                               vpart(prod[2560:4096])], 1)
        h = jax.nn.gelu(jnp.dot(xin, w1, preferred_element_type=f32) + b1)
        return jax.nn.gelu(jnp.dot(h, w2, preferred_element_type=f32) + b2) * keep

    out = pass_aq(a, charges, w11[...], b11[...], w12[...], b12[...])
    charges = nqe(charges + out[:, 0:1], out[:, 1:2])
    a = a + out[:, 2:]

    aim = pass_aq(a, charges, w21[...], b21[...], w22[...], b22[...])
    ch_out[0] = charges
    aim_out[0] = aim


def kernel(coord, numbers, charge, afv, comb_v_a, comb_v_q,
           m0_w1, m0_b1, m0_w2, m0_b2,
           m1_w1, m1_b1, m1_w2, m1_b2,
           m2_w1, m2_b1, m2_w2, m2_b2):
    f32 = jnp.float32
    coordt = coord.transpose(0, 2, 1)
    charge3 = charge.reshape(B, 1, 1)
    numbers3 = numbers.astype(jnp.int32).reshape(B, 1, N)
    w01p = _perm_w1_a(m0_w1)
    w11p = _perm_w1_aq(m1_w1)
    w21p = _perm_w1_aq(m2_w1)

    full = lambda shp: pl.BlockSpec(shp, lambda i: (0,) * len(shp))
    in_specs = [
        pl.BlockSpec((1, N, 3), lambda i: (i, 0, 0)),
        pl.BlockSpec((1, 3, N), lambda i: (i, 0, 0)),
        pl.BlockSpec((1, 1, N), lambda i: (i, 0, 0)),
        pl.BlockSpec((1, 1, 1), lambda i: (i, 0, 0)),
        full((64, NFEATURE)),
        full((NSHIFTS, NCOMB_V)),
        full((NSHIFTS, NCOMB_V)),
        full(w01p.shape), full((1, 256)), full(m0_w2.shape), full((1, 34)),
        full(w11p.shape), full((1, 256)), full(m1_w2.shape), full((1, 34)),
        full(w21p.shape), full((1, 256)), full(m2_w2.shape), full((1, 256)),
    ]
    out_specs = [
        pl.BlockSpec((1, N, 1), lambda i: (i, 0, 0)),
        pl.BlockSpec((1, N, 256), lambda i: (i, 0, 0)),
    ]
    ch, aim = pl.pallas_call(
        _fused,
        grid=(B,),
        in_specs=in_specs,
        out_specs=out_specs,
        out_shape=[jax.ShapeDtypeStruct((B, N, 1), f32),
                   jax.ShapeDtypeStruct((B, N, 256), f32)],
        compiler_params=pltpu.CompilerParams(
            dimension_semantics=("parallel",)),
    )(coord, coordt, numbers3, charge3, afv, comb_v_a, comb_v_q,
      w01p, m0_b1.reshape(1, -1), m0_w2, m0_b2.reshape(1, -1),
      w11p, m1_b1.reshape(1, -1), m1_w2, m1_b2.reshape(1, -1),
      w21p, m2_b1.reshape(1, -1), m2_w2, m2_b2.reshape(1, -1))
    return jnp.concatenate([ch, aim], -1)


# speedup vs baseline: 2.7711x; 1.4769x over previous
import jax
import jax.numpy as jnp
from jax.experimental import pallas as pl
from jax.experimental.pallas import tpu as pltpu

NFEATURE = 32
NSHIFTS = 16
NCOMB_V = 8
RC = 5.0
B, N = 128, 64
HID = 256
MB = 2


def _perm_w1_a(w1):
    a_part = w1[0:32]
    xs = w1[32:544].reshape(NFEATURE, NSHIFTS, HID).transpose(1, 0, 2)
    sv = w1[544:800].reshape(NFEATURE, NCOMB_V, HID).transpose(1, 0, 2)
    return jnp.concatenate([a_part, xs.reshape(512, HID),
                            sv.reshape(256, HID)], 0)


def _perm_w1_aq(w1):
    z1 = jnp.zeros((NCOMB_V, 1, HID), w1.dtype)
    a_part = w1[0:32]
    xs = jnp.concatenate(
        [w1[32:544].reshape(NFEATURE, NSHIFTS, HID).transpose(1, 0, 2),
         w1[801:817][:, None, :]], 1)
    sv = jnp.concatenate(
        [w1[544:800].reshape(NFEATURE, NCOMB_V, HID).transpose(1, 0, 2),
         z1], 1)
    svq = jnp.concatenate(
        [jnp.zeros((NCOMB_V, NFEATURE, HID), w1.dtype),
         w1[817:825][:, None, :]], 1)
    return jnp.concatenate([a_part, xs.reshape(528, HID), sv.reshape(264, HID),
                            w1[800:801], svq.reshape(264, HID)], 0)


def _geometry(coord_m, coordt_m, nrow, ncol):
    f32 = jnp.float32
    rij = [coord_m[:, d:d + 1] - coordt_m[d:d + 1, :] for d in range(3)]
    dist = jnp.sqrt(rij[0] * rij[0] + rij[1] * rij[1]
                    + rij[2] * rij[2] + 1e-12)
    ii = jax.lax.broadcasted_iota(jnp.int32, (N, N), 0)
    jj = jax.lax.broadcasted_iota(jnp.int32, (N, N), 1)
    valid = ((ncol != 0) & (nrow != 0) & (ii != jj) & (dist < RC))
    fc = 0.5 * jnp.cos(jnp.pi * jnp.clip(dist, 0.0, RC) / RC) + 0.5
    fc = jnp.where(valid, fc, 0.0)
    dsafe = jnp.where(valid, dist, 1.0)
    u = jnp.stack([jnp.where(valid, r / dsafe, 0.0) for r in rij], 0)

    s_io = jax.lax.broadcasted_iota(jnp.int32, (NSHIFTS, 1, 1), 0).astype(f32)
    shifts = 0.8 + s_io * ((RC - 0.8) / (NSHIFTS - 1))
    GS2 = jnp.exp(-4.0 * (dist[None] - shifts) ** 2) * fc[None]
    return GS2, u


def _fused(coord, coordt, numbers, charge, afv, cva, cvq,
           w01, b01, w02, b02, w11, b11, w12, b12, w21, b21, w22, b22,
           ch_out, aim_out):
    f32 = jnp.float32
    iota_n = jax.lax.broadcasted_iota(jnp.int32, (N, 64), 1)
    cvaT = cva[...].T
    cvqT = cvq[...].T
    afv_v = afv[...]

    LHS, a_l, keep_l = [], [], []
    for m in range(MB):
        ncol = numbers[m].T
        oh = (ncol == iota_n).astype(f32)
        a_l.append(jnp.dot(oh, afv_v, preferred_element_type=f32))
        keep_l.append(1.0 - oh[:, :1])
        GS2, u = _geometry(coord[m], coordt[m], numbers[m], ncol)
        GS2f = GS2.reshape(NSHIFTS, N * N)
        gka = jnp.dot(cvaT, GS2f, preferred_element_type=f32
                      ).reshape(NCOMB_V, N, N)
        gkq = jnp.dot(cvqT, GS2f, preferred_element_type=f32
                      ).reshape(NCOMB_V, N, N)
        ugka = u[:, None, :, :] * gka[None, :, :, :]
        ugkq = u[:, None, :, :] * gkq[None, :, :, :]
        LHS.append(jnp.concatenate([GS2.reshape(N * NSHIFTS, N),
                                    ugka.reshape(N * 24, N),
                                    ugkq.reshape(N * 24, N)], 0))

    keep = jnp.concatenate(keep_l, 0)

    def lane_blocks(mat, nblk):
        return jnp.concatenate([mat[t * N:(t + 1) * N] for t in range(nblk)], 1)

    def vpart(mat):
        sq = (mat * mat).reshape(3, NCOMB_V * N, -1).sum(0)
        return lane_blocks(sq, NCOMB_V)

    def nqe(q, f, Qm):
        w = f * f
        w = w / (w.sum(0, keepdims=True) + 1e-6)
        return q + (Qm - q.sum(0, keepdims=True)) * w

    rows = []
    for m in range(MB):
        prod = jnp.dot(LHS[m][:2560], a_l[m], preferred_element_type=f32)
        rows.append(jnp.concatenate([a_l[m], lane_blocks(prod[:1024], NSHIFTS),
                                     vpart(prod[1024:2560])], 1))
    xin = jnp.concatenate(rows, 0)
    h = jax.nn.gelu(jnp.dot(xin, w01[...], preferred_element_type=f32) + b01[...])
    out = (jnp.dot(h, w02[...], preferred_element_type=f32) + b02[...]) * keep
    ch_l = []
    for m in range(MB):
        om = out[m * N:(m + 1) * N]
        ch_l.append(nqe(om[:, 0:1], om[:, 1:2], charge[m]))
        a_l[m] = a_l[m] + om[:, 2:]

    def pass_aq(w1, b1, w2, b2):
        rows = []
        for m in range(MB):
            rhs = jnp.concatenate([a_l[m], ch_l[m]], 1)
            prod = jnp.dot(LHS[m], rhs, preferred_element_type=f32)
            rows.append(jnp.concatenate(
                [a_l[m], lane_blocks(prod[:1024], NSHIFTS),
                 vpart(prod[1024:2560]), ch_l[m],
                 vpart(prod[2560:4096])], 1))
        xin = jnp.concatenate(rows, 0)
        h = jax.nn.gelu(jnp.dot(xin, w1, preferred_element_type=f32) + b1)
        return jax.nn.gelu(jnp.dot(h, w2, preferred_element_type=f32) + b2) * keep

    out = pass_aq(w11[...], b11[...], w12[...], b12[...])
    for m in range(MB):
        om = out[m * N:(m + 1) * N]
        ch_l[m] = nqe(ch_l[m] + om[:, 0:1], om[:, 1:2], charge[m])
        a_l[m] = a_l[m] + om[:, 2:]

    aim = pass_aq(w21[...], b21[...], w22[...], b22[...])
    for m in range(MB):
        ch_out[m] = ch_l[m]
        aim_out[m] = aim[m * N:(m + 1) * N]


def kernel(coord, numbers, charge, afv, comb_v_a, comb_v_q,
           m0_w1, m0_b1, m0_w2, m0_b2,
           m1_w1, m1_b1, m1_w2, m1_b2,
           m2_w1, m2_b1, m2_w2, m2_b2):
    f32 = jnp.float32
    coordt = coord.transpose(0, 2, 1)
    charge3 = charge.reshape(B, 1, 1)
    numbers3 = numbers.astype(jnp.int32).reshape(B, 1, N)
    w01p = _perm_w1_a(m0_w1)
    w11p = _perm_w1_aq(m1_w1)
    w21p = _perm_w1_aq(m2_w1)

    full = lambda shp: pl.BlockSpec(shp, lambda i: (0,) * len(shp))
    in_specs = [
        pl.BlockSpec((MB, N, 3), lambda i: (i, 0, 0)),
        pl.BlockSpec((MB, 3, N), lambda i: (i, 0, 0)),
        pl.BlockSpec((MB, 1, N), lambda i: (i, 0, 0)),
        pl.BlockSpec((MB, 1, 1), lambda i: (i, 0, 0)),
        full((64, NFEATURE)),
        full((NSHIFTS, NCOMB_V)),
        full((NSHIFTS, NCOMB_V)),
        full(w01p.shape), full((1, 256)), full(m0_w2.shape), full((1, 34)),
        full(w11p.shape), full((1, 256)), full(m1_w2.shape), full((1, 34)),
        full(w21p.shape), full((1, 256)), full(m2_w2.shape), full((1, 256)),
    ]
    out_specs = [
        pl.BlockSpec((MB, N, 1), lambda i: (i, 0, 0)),
        pl.BlockSpec((MB, N, 256), lambda i: (i, 0, 0)),
    ]
    ch, aim = pl.pallas_call(
        _fused,
        grid=(B // MB,),
        in_specs=in_specs,
        out_specs=out_specs,
        out_shape=[jax.ShapeDtypeStruct((B, N, 1), f32),
                   jax.ShapeDtypeStruct((B, N, 256), f32)],
        compiler_params=pltpu.CompilerParams(
            dimension_semantics=("parallel",)),
    )(coord, coordt, numbers3, charge3, afv, comb_v_a, comb_v_q,
      w01p, m0_b1.reshape(1, -1), m0_w2, m0_b2.reshape(1, -1),
      w11p, m1_b1.reshape(1, -1), m1_w2, m1_b2.reshape(1, -1),
      w21p, m2_b1.reshape(1, -1), m2_w2, m2_b2.reshape(1, -1))
    return jnp.concatenate([ch, aim], -1)


# speedup vs baseline: 3.2534x; 1.1740x over previous
import jax
import jax.numpy as jnp
from jax.experimental import pallas as pl
from jax.experimental.pallas import tpu as pltpu

NFEATURE = 32
NSHIFTS = 16
NCOMB_V = 8
RC = 5.0
B, N = 128, 64
HID = 256
MB = 4


def _perm_w1_a(w1):
    a_part = w1[0:32]
    xs = w1[32:544].reshape(NFEATURE, NSHIFTS, HID).transpose(1, 0, 2)
    sv = w1[544:800].reshape(NFEATURE, NCOMB_V, HID).transpose(1, 0, 2)
    return jnp.concatenate([a_part, xs.reshape(512, HID),
                            sv.reshape(256, HID)], 0)


def _perm_w1_aq(w1):
    z1 = jnp.zeros((NCOMB_V, 1, HID), w1.dtype)
    a_part = w1[0:32]
    xs = jnp.concatenate(
        [w1[32:544].reshape(NFEATURE, NSHIFTS, HID).transpose(1, 0, 2),
         w1[801:817][:, None, :]], 1)
    sv = jnp.concatenate(
        [w1[544:800].reshape(NFEATURE, NCOMB_V, HID).transpose(1, 0, 2),
         z1], 1)
    svq = jnp.concatenate(
        [jnp.zeros((NCOMB_V, NFEATURE, HID), w1.dtype),
         w1[817:825][:, None, :]], 1)
    return jnp.concatenate([a_part, xs.reshape(528, HID), sv.reshape(264, HID),
                            w1[800:801], svq.reshape(264, HID)], 0)


def _geometry(coord_m, coordt_m, nrow, ncol):
    f32 = jnp.float32
    rij = [coord_m[:, d:d + 1] - coordt_m[d:d + 1, :] for d in range(3)]
    dist = jnp.sqrt(rij[0] * rij[0] + rij[1] * rij[1]
                    + rij[2] * rij[2] + 1e-12)
    ii = jax.lax.broadcasted_iota(jnp.int32, (N, N), 0)
    jj = jax.lax.broadcasted_iota(jnp.int32, (N, N), 1)
    valid = ((ncol != 0) & (nrow != 0) & (ii != jj) & (dist < RC))
    fc = 0.5 * jnp.cos(jnp.pi * jnp.clip(dist, 0.0, RC) / RC) + 0.5
    fc = jnp.where(valid, fc, 0.0)
    dsafe = jnp.where(valid, dist, 1.0)
    u = jnp.stack([jnp.where(valid, r / dsafe, 0.0) for r in rij], 0)

    s_io = jax.lax.broadcasted_iota(jnp.int32, (NSHIFTS, 1, 1), 0).astype(f32)
    shifts = 0.8 + s_io * ((RC - 0.8) / (NSHIFTS - 1))
    GS2 = jnp.exp(-4.0 * (dist[None] - shifts) ** 2) * fc[None]
    return GS2, u


def _fused(coord, coordt, numbers, charge, afv, cva, cvq,
           w01, b01, w02, b02, w11, b11, w12, b12, w21, b21, w22, b22,
           ch_out, aim_out):
    f32 = jnp.float32
    iota_n = jax.lax.broadcasted_iota(jnp.int32, (N, 64), 1)
    cvaT = cva[...].T
    cvqT = cvq[...].T
    afv_v = afv[...]

    LHS, a_l, keep_l = [], [], []
    for m in range(MB):
        ncol = numbers[m].T
        oh = (ncol == iota_n).astype(f32)
        a_l.append(jnp.dot(oh, afv_v, preferred_element_type=f32))
        keep_l.append(1.0 - oh[:, :1])
        GS2, u = _geometry(coord[m], coordt[m], numbers[m], ncol)
        GS2f = GS2.reshape(NSHIFTS, N * N)
        gka = jnp.dot(cvaT, GS2f, preferred_element_type=f32
                      ).reshape(NCOMB_V, N, N)
        gkq = jnp.dot(cvqT, GS2f, preferred_element_type=f32
                      ).reshape(NCOMB_V, N, N)
        ugka = u[:, None, :, :] * gka[None, :, :, :]
        ugkq = u[:, None, :, :] * gkq[None, :, :, :]
        LHS.append(jnp.concatenate([GS2.reshape(N * NSHIFTS, N),
                                    ugka.reshape(N * 24, N),
                                    ugkq.reshape(N * 24, N)], 0))

    keep = jnp.concatenate(keep_l, 0)

    def lane_blocks(mat, nblk):
        return jnp.concatenate([mat[t * N:(t + 1) * N] for t in range(nblk)], 1)

    def vpart(mat):
        sq = (mat * mat).reshape(3, NCOMB_V * N, -1).sum(0)
        return lane_blocks(sq, NCOMB_V)

    def nqe(q, f, Qm):
        w = f * f
        w = w / (w.sum(0, keepdims=True) + 1e-6)
        return q + (Qm - q.sum(0, keepdims=True)) * w

    rows = []
    for m in range(MB):
        prod = jnp.dot(LHS[m][:2560], a_l[m], preferred_element_type=f32)
        rows.append(jnp.concatenate([a_l[m], lane_blocks(prod[:1024], NSHIFTS),
                                     vpart(prod[1024:2560])], 1))
    xin = jnp.concatenate(rows, 0)
    h = jax.nn.gelu(jnp.dot(xin, w01[...], preferred_element_type=f32) + b01[...])
    out = (jnp.dot(h, w02[...], preferred_element_type=f32) + b02[...]) * keep
    ch_l = []
    for m in range(MB):
        om = out[m * N:(m + 1) * N]
        ch_l.append(nqe(om[:, 0:1], om[:, 1:2], charge[m]))
        a_l[m] = a_l[m] + om[:, 2:]

    def pass_aq(w1, b1, w2, b2):
        rows = []
        for m in range(MB):
            rhs = jnp.concatenate([a_l[m], ch_l[m]], 1)
            prod = jnp.dot(LHS[m], rhs, preferred_element_type=f32)
            rows.append(jnp.concatenate(
                [a_l[m], lane_blocks(prod[:1024], NSHIFTS),
                 vpart(prod[1024:2560]), ch_l[m],
                 vpart(prod[2560:4096])], 1))
        xin = jnp.concatenate(rows, 0)
        h = jax.nn.gelu(jnp.dot(xin, w1, preferred_element_type=f32) + b1)
        return jax.nn.gelu(jnp.dot(h, w2, preferred_element_type=f32) + b2) * keep

    out = pass_aq(w11[...], b11[...], w12[...], b12[...])
    for m in range(MB):
        om = out[m * N:(m + 1) * N]
        ch_l[m] = nqe(ch_l[m] + om[:, 0:1], om[:, 1:2], charge[m])
        a_l[m] = a_l[m] + om[:, 2:]

    aim = pass_aq(w21[...], b21[...], w22[...], b22[...])
    for m in range(MB):
        ch_out[m] = ch_l[m]
        aim_out[m] = aim[m * N:(m + 1) * N]


def kernel(coord, numbers, charge, afv, comb_v_a, comb_v_q,
           m0_w1, m0_b1, m0_w2, m0_b2,
           m1_w1, m1_b1, m1_w2, m1_b2,
           m2_w1, m2_b1, m2_w2, m2_b2):
    f32 = jnp.float32
    coordt = coord.transpose(0, 2, 1)
    charge3 = charge.reshape(B, 1, 1)
    numbers3 = numbers.astype(jnp.int32).reshape(B, 1, N)
    w01p = _perm_w1_a(m0_w1)
    w11p = _perm_w1_aq(m1_w1)
    w21p = _perm_w1_aq(m2_w1)

    full = lambda shp: pl.BlockSpec(shp, lambda i: (0,) * len(shp))
    in_specs = [
        pl.BlockSpec((MB, N, 3), lambda i: (i, 0, 0)),
        pl.BlockSpec((MB, 3, N), lambda i: (i, 0, 0)),
        pl.BlockSpec((MB, 1, N), lambda i: (i, 0, 0)),
        pl.BlockSpec((MB, 1, 1), lambda i: (i, 0, 0)),
        full((64, NFEATURE)),
        full((NSHIFTS, NCOMB_V)),
        full((NSHIFTS, NCOMB_V)),
        full(w01p.shape), full((1, 256)), full(m0_w2.shape), full((1, 34)),
        full(w11p.shape), full((1, 256)), full(m1_w2.shape), full((1, 34)),
        full(w21p.shape), full((1, 256)), full(m2_w2.shape), full((1, 256)),
    ]
    out_specs = [
        pl.BlockSpec((MB, N, 1), lambda i: (i, 0, 0)),
        pl.BlockSpec((MB, N, 256), lambda i: (i, 0, 0)),
    ]
    ch, aim = pl.pallas_call(
        _fused,
        grid=(B // MB,),
        in_specs=in_specs,
        out_specs=out_specs,
        out_shape=[jax.ShapeDtypeStruct((B, N, 1), f32),
                   jax.ShapeDtypeStruct((B, N, 256), f32)],
        compiler_params=pltpu.CompilerParams(
            dimension_semantics=("parallel",)),
    )(coord, coordt, numbers3, charge3, afv, comb_v_a, comb_v_q,
      w01p, m0_b1.reshape(1, -1), m0_w2, m0_b2.reshape(1, -1),
      w11p, m1_b1.reshape(1, -1), m1_w2, m1_b2.reshape(1, -1),
      w21p, m2_b1.reshape(1, -1), m2_w2, m2_b2.reshape(1, -1))
    return jnp.concatenate([ch, aim], -1)


# speedup vs baseline: 3.3294x; 1.0234x over previous
import jax
import jax.numpy as jnp
from jax.experimental import pallas as pl
from jax.experimental.pallas import tpu as pltpu

NFEATURE = 32
NSHIFTS = 16
NCOMB_V = 8
RC = 5.0
B, N = 128, 64
HID = 256
MB = 4


def _perm_w1_a(w1):
    a_part = w1[0:32]
    xs = w1[32:544].reshape(NFEATURE, NSHIFTS, HID).transpose(1, 0, 2)
    sv = w1[544:800].reshape(NFEATURE, NCOMB_V, HID).transpose(1, 0, 2)
    return jnp.concatenate([a_part, xs.reshape(512, HID),
                            sv.reshape(256, HID)], 0)


def _perm_w1_aq(w1):
    z1 = jnp.zeros((NCOMB_V, 1, HID), w1.dtype)
    a_part = w1[0:32]
    xs = jnp.concatenate(
        [w1[32:544].reshape(NFEATURE, NSHIFTS, HID).transpose(1, 0, 2),
         w1[801:817][:, None, :]], 1)
    sv = jnp.concatenate(
        [w1[544:800].reshape(NFEATURE, NCOMB_V, HID).transpose(1, 0, 2),
         z1], 1)
    svq = jnp.concatenate(
        [jnp.zeros((NCOMB_V, NFEATURE, HID), w1.dtype),
         w1[817:825][:, None, :]], 1)
    return jnp.concatenate([a_part, xs.reshape(528, HID), sv.reshape(264, HID),
                            w1[800:801], svq.reshape(264, HID)], 0)


def _geometry(coord_m, coordt_m, nrow, ncol):
    f32 = jnp.float32
    rij = [coord_m[:, d:d + 1] - coordt_m[d:d + 1, :] for d in range(3)]
    dist = jnp.sqrt(rij[0] * rij[0] + rij[1] * rij[1]
                    + rij[2] * rij[2] + 1e-12)
    ii = jax.lax.broadcasted_iota(jnp.int32, (N, N), 0)
    jj = jax.lax.broadcasted_iota(jnp.int32, (N, N), 1)
    valid = ((ncol != 0) & (nrow != 0) & (ii != jj) & (dist < RC))
    fc = 0.5 * jnp.cos(jnp.pi * jnp.clip(dist, 0.0, RC) / RC) + 0.5
    fc = jnp.where(valid, fc, 0.0)
    dsafe = jnp.where(valid, dist, 1.0)
    u = jnp.stack([jnp.where(valid, r / dsafe, 0.0) for r in rij], 0)

    s_io = jax.lax.broadcasted_iota(jnp.int32, (NSHIFTS, 1, 1), 0).astype(f32)
    shifts = 0.8 + s_io * ((RC - 0.8) / (NSHIFTS - 1))
    GS2 = jnp.exp(-4.0 * (dist[None] - shifts) ** 2) * fc[None]
    return GS2, u


def _fused(coord, coordt, numbers, charge, afv, cva, cvq,
           w01, b01, w02, b02, w11, b11, w12, b12, w21, b21, w22, b22,
           ch_out, aim_out):
    f32 = jnp.float32
    iota_n = jax.lax.broadcasted_iota(jnp.int32, (N, 64), 1)
    cvaT = cva[...].T
    cvqT = cvq[...].T
    afv_v = afv[...]

    LHS, a_l, keep_l = [], [], []
    for m in range(MB):
        ncol = numbers[m].T
        oh = (ncol == iota_n).astype(f32)
        a_l.append(jnp.dot(oh, afv_v, preferred_element_type=f32))
        keep_l.append(1.0 - oh[:, :1])
        GS2, u = _geometry(coord[m], coordt[m], numbers[m], ncol)
        GS2f = GS2.reshape(NSHIFTS, N * N)
        gka = jnp.dot(cvaT, GS2f, preferred_element_type=f32
                      ).reshape(NCOMB_V, N, N)
        gkq = jnp.dot(cvqT, GS2f, preferred_element_type=f32
                      ).reshape(NCOMB_V, N, N)
        ugka = u[:, None, :, :] * gka[None, :, :, :]
        ugkq = u[:, None, :, :] * gkq[None, :, :, :]
        LHS.append(jnp.concatenate([GS2.reshape(N * NSHIFTS, N),
                                    ugka.reshape(N * 24, N),
                                    ugkq.reshape(N * 24, N)],
                                   0).astype(jnp.bfloat16))

    keep = jnp.concatenate(keep_l, 0)

    def lane_blocks(mat, nblk):
        return jnp.concatenate([mat[t * N:(t + 1) * N] for t in range(nblk)], 1)

    def vpart(mat):
        sq = (mat * mat).reshape(3, NCOMB_V * N, -1).sum(0)
        return lane_blocks(sq, NCOMB_V)

    def nqe(q, f, Qm):
        w = f * f
        w = w / (w.sum(0, keepdims=True) + 1e-6)
        return q + (Qm - q.sum(0, keepdims=True)) * w

    rows = []
    for m in range(MB):
        prod = jnp.dot(LHS[m][:2560], a_l[m].astype(jnp.bfloat16),
                       preferred_element_type=f32)
        rows.append(jnp.concatenate([a_l[m], lane_blocks(prod[:1024], NSHIFTS),
                                     vpart(prod[1024:2560])], 1))
    xin = jnp.concatenate(rows, 0)
    h = jax.nn.gelu(jnp.dot(xin, w01[...], preferred_element_type=f32) + b01[...])
    out = (jnp.dot(h, w02[...], preferred_element_type=f32) + b02[...]) * keep
    ch_l = []
    for m in range(MB):
        om = out[m * N:(m + 1) * N]
        ch_l.append(nqe(om[:, 0:1], om[:, 1:2], charge[m]))
        a_l[m] = a_l[m] + om[:, 2:]

    def pass_aq(w1, b1, w2, b2):
        rows = []
        for m in range(MB):
            rhs = jnp.concatenate([a_l[m], ch_l[m]], 1).astype(jnp.bfloat16)
            prod = jnp.dot(LHS[m], rhs, preferred_element_type=f32)
            rows.append(jnp.concatenate(
                [a_l[m], lane_blocks(prod[:1024], NSHIFTS),
                 vpart(prod[1024:2560]), ch_l[m],
                 vpart(prod[2560:4096])], 1))
        xin = jnp.concatenate(rows, 0)
        h = jax.nn.gelu(jnp.dot(xin, w1, preferred_element_type=f32) + b1)
        return jax.nn.gelu(jnp.dot(h, w2, preferred_element_type=f32) + b2) * keep

    out = pass_aq(w11[...], b11[...], w12[...], b12[...])
    for m in range(MB):
        om = out[m * N:(m + 1) * N]
        ch_l[m] = nqe(ch_l[m] + om[:, 0:1], om[:, 1:2], charge[m])
        a_l[m] = a_l[m] + om[:, 2:]

    aim = pass_aq(w21[...], b21[...], w22[...], b22[...])
    for m in range(MB):
        ch_out[m] = ch_l[m]
        aim_out[m] = aim[m * N:(m + 1) * N]


def kernel(coord, numbers, charge, afv, comb_v_a, comb_v_q,
           m0_w1, m0_b1, m0_w2, m0_b2,
           m1_w1, m1_b1, m1_w2, m1_b2,
           m2_w1, m2_b1, m2_w2, m2_b2):
    f32 = jnp.float32
    coordt = coord.transpose(0, 2, 1)
    charge3 = charge.reshape(B, 1, 1)
    numbers3 = numbers.astype(jnp.int32).reshape(B, 1, N)
    w01p = _perm_w1_a(m0_w1)
    w11p = _perm_w1_aq(m1_w1)
    w21p = _perm_w1_aq(m2_w1)

    full = lambda shp: pl.BlockSpec(shp, lambda i: (0,) * len(shp))
    in_specs = [
        pl.BlockSpec((MB, N, 3), lambda i: (i, 0, 0)),
        pl.BlockSpec((MB, 3, N), lambda i: (i, 0, 0)),
        pl.BlockSpec((MB, 1, N), lambda i: (i, 0, 0)),
        pl.BlockSpec((MB, 1, 1), lambda i: (i, 0, 0)),
        full((64, NFEATURE)),
        full((NSHIFTS, NCOMB_V)),
        full((NSHIFTS, NCOMB_V)),
        full(w01p.shape), full((1, 256)), full(m0_w2.shape), full((1, 34)),
        full(w11p.shape), full((1, 256)), full(m1_w2.shape), full((1, 34)),
        full(w21p.shape), full((1, 256)), full(m2_w2.shape), full((1, 256)),
    ]
    out_specs = [
        pl.BlockSpec((MB, N, 1), lambda i: (i, 0, 0)),
        pl.BlockSpec((MB, N, 256), lambda i: (i, 0, 0)),
    ]
    ch, aim = pl.pallas_call(
        _fused,
        grid=(B // MB,),
        in_specs=in_specs,
        out_specs=out_specs,
        out_shape=[jax.ShapeDtypeStruct((B, N, 1), f32),
                   jax.ShapeDtypeStruct((B, N, 256), f32)],
        compiler_params=pltpu.CompilerParams(
            dimension_semantics=("parallel",)),
    )(coord, coordt, numbers3, charge3, afv, comb_v_a, comb_v_q,
      w01p, m0_b1.reshape(1, -1), m0_w2, m0_b2.reshape(1, -1),
      w11p, m1_b1.reshape(1, -1), m1_w2, m1_b2.reshape(1, -1),
      w21p, m2_b1.reshape(1, -1), m2_w2, m2_b2.reshape(1, -1))
    return jnp.concatenate([ch, aim], -1)


# speedup vs baseline: 3.4790x; 1.0449x over previous
import jax
import jax.numpy as jnp
from jax.experimental import pallas as pl
from jax.experimental.pallas import tpu as pltpu

NFEATURE = 32
NSHIFTS = 16
NCOMB_V = 8
RC = 5.0
B, N = 128, 64
HID = 256
MB = 4


def _perm_w1_a(w1):
    a_part = w1[0:32]
    xs = w1[32:544].reshape(NFEATURE, NSHIFTS, HID).transpose(1, 0, 2)
    sv = w1[544:800].reshape(NFEATURE, NCOMB_V, HID).transpose(1, 0, 2)
    return jnp.concatenate([a_part, xs.reshape(512, HID),
                            sv.reshape(256, HID)], 0)


def _perm_w1_aq(w1):
    z1 = jnp.zeros((NCOMB_V, 1, HID), w1.dtype)
    a_part = w1[0:32]
    xs = jnp.concatenate(
        [w1[32:544].reshape(NFEATURE, NSHIFTS, HID).transpose(1, 0, 2),
         w1[801:817][:, None, :]], 1)
    sv = jnp.concatenate(
        [w1[544:800].reshape(NFEATURE, NCOMB_V, HID).transpose(1, 0, 2),
         z1], 1)
    svq = jnp.concatenate(
        [jnp.zeros((NCOMB_V, NFEATURE, HID), w1.dtype),
         w1[817:825][:, None, :]], 1)
    return jnp.concatenate([a_part, xs.reshape(528, HID), sv.reshape(264, HID),
                            w1[800:801], svq.reshape(264, HID)], 0)


def _geometry(coord_m, coordt_m, nrow, ncol):
    f32 = jnp.float32
    rij = [coord_m[:, d:d + 1] - coordt_m[d:d + 1, :] for d in range(3)]
    dist = jnp.sqrt(rij[0] * rij[0] + rij[1] * rij[1]
                    + rij[2] * rij[2] + 1e-12)
    ii = jax.lax.broadcasted_iota(jnp.int32, (N, N), 0)
    jj = jax.lax.broadcasted_iota(jnp.int32, (N, N), 1)
    valid = ((ncol != 0) & (nrow != 0) & (ii != jj) & (dist < RC))
    fc = 0.5 * jnp.cos(jnp.pi * jnp.clip(dist, 0.0, RC) / RC) + 0.5
    fc = jnp.where(valid, fc, 0.0)
    dsafe = jnp.where(valid, dist, 1.0)
    u = jnp.stack([jnp.where(valid, r / dsafe, 0.0) for r in rij], 0)

    s_io = jax.lax.broadcasted_iota(jnp.int32, (NSHIFTS, 1, 1), 0).astype(f32)
    shifts = 0.8 + s_io * ((RC - 0.8) / (NSHIFTS - 1))
    GS2 = jnp.exp(-4.0 * (dist[None] - shifts) ** 2) * fc[None]
    return GS2, u


def _fused(coord, coordt, numbers, charge, afv, cva, cvq,
           w01, b01, w02, b02, w11, b11, w12, b12, w21, b21, w22, b22,
           ch_out, aim_out):
    f32 = jnp.float32
    iota_n = jax.lax.broadcasted_iota(jnp.int32, (N, 64), 1)
    cvaT = cva[...].T
    cvqT = cvq[...].T
    afv_v = afv[...]

    LHS, a_l, keep_l = [], [], []
    for m in range(MB):
        ncol = numbers[m].T
        oh = (ncol == iota_n).astype(f32)
        a_l.append(jnp.dot(oh, afv_v, preferred_element_type=f32))
        keep_l.append(1.0 - oh[:, :1])
        GS2, u = _geometry(coord[m], coordt[m], numbers[m], ncol)
        GS2f = GS2.reshape(NSHIFTS, N * N)
        gka = jnp.dot(cvaT, GS2f, preferred_element_type=f32
                      ).reshape(NCOMB_V, N, N)
        gkq = jnp.dot(cvqT, GS2f, preferred_element_type=f32
                      ).reshape(NCOMB_V, N, N)
        ugka = u[:, None, :, :] * gka[None, :, :, :]
        ugkq = u[:, None, :, :] * gkq[None, :, :, :]
        LHS.append(jnp.concatenate([GS2.reshape(N * NSHIFTS, N),
                                    ugka.reshape(N * 24, N),
                                    ugkq.reshape(N * 24, N)],
                                   0).astype(jnp.bfloat16))

    keep = jnp.concatenate(keep_l, 0)

    def lane_blocks(mat, nblk):
        return jnp.concatenate([mat[t * N:(t + 1) * N] for t in range(nblk)], 1)

    def vpart(mat):
        sq = (mat * mat).reshape(3, NCOMB_V * N, -1).sum(0)
        return lane_blocks(sq, NCOMB_V)

    def nqe(q, f, Qm):
        w = f * f
        w = w / (w.sum(0, keepdims=True) + 1e-6)
        return q + (Qm - q.sum(0, keepdims=True)) * w

    rows = []
    for m in range(MB):
        prod = jnp.dot(LHS[m][:2560], a_l[m].astype(jnp.bfloat16),
                       preferred_element_type=f32)
        rows.append(jnp.concatenate([a_l[m], lane_blocks(prod[:1024], NSHIFTS),
                                     vpart(prod[1024:2560])], 1))
    xin = jnp.concatenate(rows, 0)
    h = jax.nn.gelu(jnp.dot(xin.astype(jnp.bfloat16), w01[...],
                            preferred_element_type=f32) + b01[...])
    out = (jnp.dot(h.astype(jnp.bfloat16), w02[...],
                   preferred_element_type=f32) + b02[...]) * keep
    ch_l = []
    for m in range(MB):
        om = out[m * N:(m + 1) * N]
        ch_l.append(nqe(om[:, 0:1], om[:, 1:2], charge[m]))
        a_l[m] = a_l[m] + om[:, 2:]

    def pass_aq(w1, b1, w2, b2):
        rows = []
        for m in range(MB):
            rhs = jnp.concatenate([a_l[m], ch_l[m]], 1).astype(jnp.bfloat16)
            prod = jnp.dot(LHS[m], rhs, preferred_element_type=f32)
            rows.append(jnp.concatenate(
                [a_l[m], lane_blocks(prod[:1024], NSHIFTS),
                 vpart(prod[1024:2560]), ch_l[m],
                 vpart(prod[2560:4096])], 1))
        xin = jnp.concatenate(rows, 0)
        h = jax.nn.gelu(jnp.dot(xin.astype(jnp.bfloat16), w1,
                                preferred_element_type=f32) + b1)
        return jax.nn.gelu(jnp.dot(h.astype(jnp.bfloat16), w2,
                                   preferred_element_type=f32) + b2) * keep

    out = pass_aq(w11[...], b11[...], w12[...], b12[...])
    for m in range(MB):
        om = out[m * N:(m + 1) * N]
        ch_l[m] = nqe(ch_l[m] + om[:, 0:1], om[:, 1:2], charge[m])
        a_l[m] = a_l[m] + om[:, 2:]

    aim = pass_aq(w21[...], b21[...], w22[...], b22[...])
    for m in range(MB):
        ch_out[m] = ch_l[m]
        aim_out[m] = aim[m * N:(m + 1) * N]


def kernel(coord, numbers, charge, afv, comb_v_a, comb_v_q,
           m0_w1, m0_b1, m0_w2, m0_b2,
           m1_w1, m1_b1, m1_w2, m1_b2,
           m2_w1, m2_b1, m2_w2, m2_b2):
    f32 = jnp.float32
    coordt = coord.transpose(0, 2, 1)
    charge3 = charge.reshape(B, 1, 1)
    numbers3 = numbers.astype(jnp.int32).reshape(B, 1, N)
    w01p = _perm_w1_a(m0_w1)
    w11p = _perm_w1_aq(m1_w1)
    w21p = _perm_w1_aq(m2_w1)

    full = lambda shp: pl.BlockSpec(shp, lambda i: (0,) * len(shp))
    in_specs = [
        pl.BlockSpec((MB, N, 3), lambda i: (i, 0, 0)),
        pl.BlockSpec((MB, 3, N), lambda i: (i, 0, 0)),
        pl.BlockSpec((MB, 1, N), lambda i: (i, 0, 0)),
        pl.BlockSpec((MB, 1, 1), lambda i: (i, 0, 0)),
        full((64, NFEATURE)),
        full((NSHIFTS, NCOMB_V)),
        full((NSHIFTS, NCOMB_V)),
        full(w01p.shape), full((1, 256)), full(m0_w2.shape), full((1, 34)),
        full(w11p.shape), full((1, 256)), full(m1_w2.shape), full((1, 34)),
        full(w21p.shape), full((1, 256)), full(m2_w2.shape), full((1, 256)),
    ]
    out_specs = [
        pl.BlockSpec((MB, N, 1), lambda i: (i, 0, 0)),
        pl.BlockSpec((MB, N, 256), lambda i: (i, 0, 0)),
    ]
    ch, aim = pl.pallas_call(
        _fused,
        grid=(B // MB,),
        in_specs=in_specs,
        out_specs=out_specs,
        out_shape=[jax.ShapeDtypeStruct((B, N, 1), f32),
                   jax.ShapeDtypeStruct((B, N, 256), f32)],
        compiler_params=pltpu.CompilerParams(
            dimension_semantics=("parallel",)),
    )(coord, coordt, numbers3, charge3, afv, comb_v_a, comb_v_q,
      w01p.astype(jnp.bfloat16), m0_b1.reshape(1, -1),
      m0_w2.astype(jnp.bfloat16), m0_b2.reshape(1, -1),
      w11p.astype(jnp.bfloat16), m1_b1.reshape(1, -1),
      m1_w2.astype(jnp.bfloat16), m1_b2.reshape(1, -1),
      w21p.astype(jnp.bfloat16), m2_b1.reshape(1, -1),
      m2_w2.astype(jnp.bfloat16), m2_b2.reshape(1, -1))
    return jnp.concatenate([ch, aim], -1)
